# Initial kernel scaffold; baseline (speedup 1.0000x reference)
import jax
import jax.numpy as jnp
from jax.experimental import pallas as pl


def kernel(x_h, x_t, edge_index_h, edge_index_t, batch_h, batch_t, rels, params):
    raise NotImplementedError("write your pallas kernel here")


# speedup vs baseline: 2.7656x; 2.7656x over previous
import functools
from functools import partial

import jax
import jax.numpy as jnp
from jax import lax
from jax.experimental import pallas as pl
from jax.experimental.pallas import tpu as pltpu
from jax.experimental.pallas import tpu_sc as plsc

_N = 10000
_E = 160000
_H = 128
_L = 4
_G = 256
_R = 86

_N2 = 2 * _N
_G2 = 2 * _G
_NB = 800
_NBLK = _N2 // _NB

_NTILES = 32
_EPT = 2 * _E // _NTILES
_CH = 125
_NCHUNK = _EPT // _CH
_STRIPE = 624


def _segsum_body(x_hbm, src_hbm, dst_hbm, out_hbm,
                 src_v, dst_v, rows0, sem0, acc_sh):
    c = lax.axis_index("c")
    s = lax.axis_index("s")
    wid = c * 16 + s

    pltpu.sync_copy(src_hbm.at[wid], src_v)
    pltpu.sync_copy(dst_hbm.at[wid], dst_v)

    def _with_zbuf(zb):
        def _zero_row(i, _):
            for j in range(_H // 16):
                zb[i, pl.ds(j * 16, 16)] = jnp.zeros((16,), jnp.float32)
            return 0

        lax.fori_loop(0, 80, _zero_row, 0)
        for r in range(7):
            pltpu.sync_copy(zb, acc_sh.at[pl.ds(s * _STRIPE + r * 80, 80)])
        pltpu.sync_copy(zb.at[pl.ds(0, 64)],
                        acc_sh.at[pl.ds(s * _STRIPE + 560, 64)])

        @pl.when(s == 15)
        def _():
            pltpu.sync_copy(zb.at[pl.ds(0, 16)],
                            acc_sh.at[pl.ds(16 * _STRIPE, 16)])

    pl.run_scoped(_with_zbuf, pltpu.VMEM((80, _H), jnp.float32))
    plsc.subcore_barrier()

    def _step(i, _):
        pltpu.async_copy(x_hbm.at[src_v.at[i]], rows0, sem0).wait()
        pltpu.sync_copy(rows0, acc_sh.at[dst_v.at[i]], add=True)
        return 0

    lax.fori_loop(0, _NCHUNK, _step, 0)
    plsc.subcore_barrier()

    pltpu.sync_copy(acc_sh.at[pl.ds(s * _STRIPE, _STRIPE)],
                    out_hbm.at[pl.ds(c * _N + s * _STRIPE, _STRIPE)])

    @pl.when(s == 15)
    def _():
        pltpu.sync_copy(acc_sh.at[pl.ds(16 * _STRIPE, _N - 16 * _STRIPE)],
                        out_hbm.at[pl.ds(c * _N + 16 * _STRIPE,
                                         _N - 16 * _STRIPE)])


def _segsum_sc(x, src3, dst3):
    mesh = plsc.VectorSubcoreMesh(core_axis_name="c", subcore_axis_name="s")
    return pl.kernel(
        _segsum_body,
        out_type=jax.ShapeDtypeStruct((_N2, _H), jnp.float32),
        mesh=mesh,
        scratch_types=[
            pltpu.VMEM((_NCHUNK, _CH), jnp.int32),
            pltpu.VMEM((_NCHUNK, _CH), jnp.int32),
            pltpu.VMEM((_CH, _H), jnp.float32),
            pltpu.SemaphoreType.DMA,
            pltpu.VMEM_SHARED((_N, _H), jnp.float32),
        ],
    )(x, src3, dst3)


def _stats_kernel(x_ref, bf_ref, out_ref):
    i = pl.program_id(0)
    x = x_ref[...]
    bf = bf_ref[0]
    gids = lax.broadcasted_iota(jnp.int32, (_G2, _NB), 0)
    mask = (gids == bf.astype(jnp.int32)).astype(jnp.float32)
    lanes = lax.broadcasted_iota(jnp.int32, (_NB, _H), 1)
    rs = jnp.sum(x, axis=1, keepdims=True)
    sq = jnp.sum(x * x, axis=1, keepdims=True)
    feat = jnp.where(lanes == 0, 1.0,
                     jnp.where(lanes == 1, rs, jnp.where(lanes == 2, sq, 0.0)))

    @pl.when(i == 0)
    def _():
        out_ref[...] = jnp.zeros_like(out_ref)

    out_ref[...] += jnp.dot(mask, feat, preferred_element_type=jnp.float32, precision=lax.Precision.HIGHEST)


def _lnapply_kernel(x_ref, bf_ref, stats_ref, w0_ref, aux_ref, y_ref):
    st = stats_ref[...]
    cnt = jnp.maximum(st[:, 0:1] * _H, 1.0)
    mean = st[:, 1:2] / cnt
    var = st[:, 2:3] / cnt - mean * mean
    rstd = lax.rsqrt(var + 1e-5)
    lanes = lax.broadcasted_iota(jnp.int32, (_G2, _H), 1)
    derived = jnp.where(lanes == 0, mean, jnp.where(lanes == 1, rstd, 0.0))

    bf = bf_ref[0]
    gids = lax.broadcasted_iota(jnp.int32, (_G2, _NB), 0)
    mask = (gids == bf.astype(jnp.int32)).astype(jnp.float32)
    nodevals = lax.dot_general(mask, derived, (((0,), (0,)), ((), ())),
                               preferred_element_type=jnp.float32, precision=lax.Precision.HIGHEST)
    mn = nodevals[:, 0:1]
    rstd_n = nodevals[:, 1:2]

    x = x_ref[...]
    aux = aux_ref[...]
    xln = (x - mn) * rstd_n * aux[0:1, :] + aux[1:2, :]
    y_ref[...] = (jnp.dot(xln, w0_ref[...], preferred_element_type=jnp.float32, precision=lax.Precision.HIGHEST)
                  + aux[2:3, :])


def _layerA_kernel(xi_ref, agg_ref, x_ref, wm1_ref, wm2_ref, aux_ref,
                   xm_ref, maskb_ref):
    aux = aux_ref[...]
    h = jnp.dot(xi_ref[...], wm1_ref[...], preferred_element_type=jnp.float32, precision=lax.Precision.HIGHEST)
    h += jnp.dot(agg_ref[...], wm2_ref[...], preferred_element_type=jnp.float32, precision=lax.Precision.HIGHEST)
    h = jax.nn.relu(h + aux[0:1, :])
    hv = jnp.sum(h * aux[1:2, :], axis=1, keepdims=True)
    m = jax.nn.sigmoid(hv + aux[2:3, 0:1])
    mb = jnp.broadcast_to(m, (_NB, _H))
    xm_ref[...] = x_ref[...] * mb
    maskb_ref[...] = mb


def _layerB_kernel(xm_ref, agg2_ref, maskb_ref, bf_ref, wr_ref, wn_ref,
                   aux_ref, xn_ref, xi_ref, pool_ref):
    i = pl.program_id(0)
    xn = jnp.dot(xm_ref[...], wr_ref[...], preferred_element_type=jnp.float32, precision=lax.Precision.HIGHEST)
    xn += jnp.dot(agg2_ref[...], wn_ref[...], preferred_element_type=jnp.float32, precision=lax.Precision.HIGHEST)
    xn = jax.nn.relu(xn + aux_ref[0:1, :])
    xn_ref[...] = xn
    xi_ref[...] = xn * maskb_ref[...]

    bf = bf_ref[0]
    gids = lax.broadcasted_iota(jnp.int32, (_G2, _NB), 0)
    mask = (gids == bf.astype(jnp.int32)).astype(jnp.float32)

    @pl.when(i == 0)
    def _():
        pool_ref[...] = jnp.zeros_like(pool_ref)

    pool_ref[...] += jnp.dot(mask, xn, preferred_element_type=jnp.float32, precision=lax.Precision.HIGHEST)


def _zmlp_kernel(p_ref, w1_ref, b1_ref, w2_ref, b2_ref, z_ref):
    z = jax.nn.relu(jnp.dot(p_ref[...], w1_ref[...],
                            preferred_element_type=jnp.float32, precision=lax.Precision.HIGHEST) + b1_ref[...])
    z_ref[...] = jax.nn.relu(jnp.dot(z, w2_ref[...],
                                     preferred_element_type=jnp.float32, precision=lax.Precision.HIGHEST)
                             + b2_ref[...])


def _coatt_kernel(rels_ref, kh_ref, kt_ref, r_ref, wq_ref, wk_ref, cb_ref,
                  ca_ref, wx_ref, bx_ref, wy_ref, by_ref, wa_ref, ba_ref,
                  out_ref):
    khg = kh_ref[0]
    ktg = kt_ref[0]
    keys = jnp.dot(khg, wk_ref[...], preferred_element_type=jnp.float32, precision=lax.Precision.HIGHEST)
    queries = jnp.dot(ktg, wq_ref[...], preferred_element_type=jnp.float32, precision=lax.Precision.HIGHEST)
    cb = cb_ref[...]
    ca = ca_ref[...]

    cols = []
    for i in range(_L):
        e_i = queries + keys[i:i + 1, :] + cb
        t = jnp.tanh(e_i)
        cols.append(jnp.dot(t, ca, preferred_element_type=jnp.float32, precision=lax.Precision.HIGHEST))
    attnT = jnp.concatenate(cols, axis=1)

    daxT = lax.dot_general(wx_ref[...], khg, (((0,), (0,)), ((), ())),
                           preferred_element_type=jnp.float32, precision=lax.Precision.HIGHEST) + bx_ref[...]
    dayT = lax.dot_general(wy_ref[...], ktg, (((0,), (0,)), ((), ())),
                           preferred_element_type=jnp.float32, precision=lax.Precision.HIGHEST) + by_ref[...]

    ones = jnp.ones((1, _H), jnp.float32)
    mxt_rows = []
    myt_rows = []
    for l in range(_L):
        rx = daxT[l:l + 1, :]
        ry = dayT[l:l + 1, :]
        colmat = lax.dot_general(rx, ones, (((0,), (0,)), ((), ())),
                                 preferred_element_type=jnp.float32, precision=lax.Precision.HIGHEST)
        rowmat = lax.dot_general(ones, ry, (((0,), (0,)), ((), ())),
                                 preferred_element_type=jnp.float32, precision=lax.Precision.HIGHEST)
        p_l = jax.nn.relu(colmat + rowmat)
        mxt_rows.append(lax.dot_general(
            ones, p_l, (((1,), (1,)), ((), ())),
            preferred_element_type=jnp.float32, precision=lax.Precision.HIGHEST) * (1.0 / _H))
        myt_rows.append(jnp.dot(ones, p_l,
                                preferred_element_type=jnp.float32, precision=lax.Precision.HIGHEST) * (1.0 / _H))
    mxT = jnp.concatenate(mxt_rows, axis=0)
    myT = jnp.concatenate(myt_rows, axis=0)

    cx = jax.nn.sigmoid(lax.dot_general(wa_ref[...], mxT, (((0,), (0,)), ((), ())),
                                        preferred_element_type=jnp.float32, precision=lax.Precision.HIGHEST)
                        + ba_ref[...])
    cy = jax.nn.sigmoid(lax.dot_general(wa_ref[...], myT, (((0,), (0,)), ((), ())),
                                        preferred_element_type=jnp.float32, precision=lax.Precision.HIGHEST)
                        + ba_ref[...])

    kh = khg * (0.5 + cx)
    kt = ktg * (0.5 + cy)
    hn = jnp.maximum(jnp.sqrt(jnp.sum(kh * kh, axis=1, keepdims=True)), 1e-12)
    tn = jnp.maximum(jnp.sqrt(jnp.sum(kt * kt, axis=1, keepdims=True)), 1e-12)
    heads = kh / hn
    tails = kt / tn

    r = r_ref[0]
    s1t = lax.dot_general(tails, r, (((1,), (1,)), ((), ())),
                          preferred_element_type=jnp.float32, precision=lax.Precision.HIGHEST)
    valT = lax.dot_general(s1t, heads, (((1,), (1,)), ((), ())),
                           preferred_element_type=jnp.float32, precision=lax.Precision.HIGHEST)
    score = jnp.sum(attnT * valT)
    out_ref[...] = jnp.broadcast_to(score, (1, 1, _H))


def _row_spec(i_map=None):
    return pl.BlockSpec((_NB, _H), i_map or (lambda i: (i, 0)))


def _const_spec(shape):
    return pl.BlockSpec(shape, lambda i: tuple(0 for _ in shape))


def kernel(x_h, x_t, edge_index_h, edge_index_t, batch_h, batch_t, rels, params):
    f32 = jnp.float32
    x = jnp.concatenate([x_h, x_t], axis=0)
    batchf = jnp.concatenate([batch_h, batch_t + _G]).astype(f32)
    batchf = batchf.reshape(_NBLK, 1, _NB)

    src3 = jnp.concatenate([edge_index_h[0], edge_index_t[0] + _N]
                           ).reshape(_NTILES, _NCHUNK, _CH)
    dst3 = jnp.concatenate([edge_index_h[1], edge_index_t[1]]
                           ).reshape(_NTILES, _NCHUNK, _CH)

    p = params
    ln_aux = jnp.zeros((8, _H), f32)
    ln_aux = ln_aux.at[0].set(p['ln_w']).at[1].set(p['ln_b'])
    ln_aux = ln_aux.at[2].set(p['lin0'][1])
    w0 = p['lin0'][0]

    bspec = pl.BlockSpec((1, 1, _NB), lambda i: (i, 0, 0))

    stats = pl.pallas_call(
        _stats_kernel,
        grid=(_NBLK,),
        in_specs=[_row_spec(), bspec],
        out_specs=_const_spec((_G2, _H)),
        out_shape=jax.ShapeDtypeStruct((_G2, _H), f32),
    )(x, batchf)

    y = pl.pallas_call(
        _lnapply_kernel,
        grid=(_NBLK,),
        in_specs=[_row_spec(), bspec, _const_spec((_G2, _H)),
                  _const_spec((_H, _H)), _const_spec((8, _H))],
        out_specs=_row_spec(),
        out_shape=jax.ShapeDtypeStruct((_N2, _H), f32),
    )(x, batchf, stats, w0, ln_aux)

    xcur = y
    xi = y
    pooled = []
    for i in range(_L):
        (wm1, bm1), (wm2, bm2), (wm3, bm3) = p['masks'][i]
        (wr, br), (wn, bn) = p['convs'][i]
        auxA = jnp.zeros((8, _H), f32)
        auxA = auxA.at[0].set(bm1 + bm2).at[1].set(wm3[:, 0]).at[2].set(bm3[0])
        auxB = jnp.zeros((8, _H), f32).at[0].set(br + bn)

        agg = _segsum_sc(xi, src3, dst3)
        xm, maskb = pl.pallas_call(
            _layerA_kernel,
            grid=(_NBLK,),
            in_specs=[_row_spec(), _row_spec(), _row_spec(),
                      _const_spec((_H, _H)), _const_spec((_H, _H)),
                      _const_spec((8, _H))],
            out_specs=[_row_spec(), _row_spec()],
            out_shape=[jax.ShapeDtypeStruct((_N2, _H), f32),
                       jax.ShapeDtypeStruct((_N2, _H), f32)],
        )(xi, agg, xcur, wm1, wm2, auxA)

        agg2 = _segsum_sc(xm, src3, dst3)
        xcur, xi, pool_i = pl.pallas_call(
            _layerB_kernel,
            grid=(_NBLK,),
            in_specs=[_row_spec(), _row_spec(), _row_spec(), bspec,
                      _const_spec((_H, _H)), _const_spec((_H, _H)),
                      _const_spec((8, _H))],
            out_specs=[_row_spec(), _row_spec(), _const_spec((_G2, _H))],
            out_shape=[jax.ShapeDtypeStruct((_N2, _H), f32),
                       jax.ShapeDtypeStruct((_N2, _H), f32),
                       jax.ShapeDtypeStruct((_G2, _H), f32)],
        )(xm, agg2, maskb, batchf, wr, wn, auxB)
        pooled.append(pool_i)

    pall = jnp.stack(pooled, axis=0).reshape(_L * _G2, _H)
    w1, b1 = p['lin1']
    w2, b2 = p['lin2']
    z = pl.pallas_call(
        _zmlp_kernel,
        grid=(1,),
        in_specs=[_const_spec((_L * _G2, _H)), _const_spec((_H, 2 * _H)),
                  _const_spec((1, 2 * _H)), _const_spec((2 * _H, _H)),
                  _const_spec((1, _H))],
        out_specs=_const_spec((_L * _G2, _H)),
        out_shape=jax.ShapeDtypeStruct((_L * _G2, _H), f32),
    )(pall, w1, b1.reshape(1, -1), w2, b2.reshape(1, -1))

    z = z.reshape(_L, _G2, _H)
    kge_h = jnp.transpose(z[:, :_G], (1, 0, 2))
    kge_t = jnp.transpose(z[:, _G:], (1, 0, 2))

    rel3 = p['rel_emb'].reshape(_R, _H, _H)
    wxa, bxa = p['att_x']
    wya, bya = p['att_y']
    waa, baa = p['att']

    grid_spec = pltpu.PrefetchScalarGridSpec(
        num_scalar_prefetch=1,
        grid=(_G,),
        in_specs=[
            pl.BlockSpec((1, _L, _H), lambda g, r_: (g, 0, 0)),
            pl.BlockSpec((1, _L, _H), lambda g, r_: (g, 0, 0)),
            pl.BlockSpec((1, _H, _H), lambda g, r_: (r_[g], 0, 0)),
            pl.BlockSpec((_H, _H // 2), lambda g, r_: (0, 0)),
            pl.BlockSpec((_H, _H // 2), lambda g, r_: (0, 0)),
            pl.BlockSpec((1, _H // 2), lambda g, r_: (0, 0)),
            pl.BlockSpec((_H // 2, 1), lambda g, r_: (0, 0)),
            pl.BlockSpec((_L, _L), lambda g, r_: (0, 0)),
            pl.BlockSpec((_L, 1), lambda g, r_: (0, 0)),
            pl.BlockSpec((_L, _L), lambda g, r_: (0, 0)),
            pl.BlockSpec((_L, 1), lambda g, r_: (0, 0)),
            pl.BlockSpec((_L, _L), lambda g, r_: (0, 0)),
            pl.BlockSpec((_L, 1), lambda g, r_: (0, 0)),
        ],
        out_specs=pl.BlockSpec((1, 1, _H), lambda g, r_: (g, 0, 0)),
    )
    out = pl.pallas_call(
        _coatt_kernel,
        grid_spec=grid_spec,
        out_shape=jax.ShapeDtypeStruct((_G, 1, _H), f32),
    )(rels, kge_h, kge_t, rel3, p['co_wq'], p['co_wk'],
      p['co_b'].reshape(1, -1), p['co_a'].reshape(-1, 1),
      wxa, bxa.reshape(-1, 1), wya, bya.reshape(-1, 1),
      waa, baa.reshape(-1, 1))
    return out[:, 0, 0]
